# baseline (speedup 1.0000x reference)
import functools

import jax
import jax.numpy as jnp
from jax import lax
from jax.experimental import pallas as pl
from jax.experimental.pallas import tpu as pltpu
from jax.experimental.pallas import tpu_sc as plsc

D_MODEL = 64
NUM_WORKERS = 32
BATCH_CHUNK = 2


@functools.lru_cache(maxsize=None)
def _make_gather(batch: int, hist: int):
    assert batch % (NUM_WORKERS * BATCH_CHUNK) == 0
    rows = BATCH_CHUNK * hist
    batches_per_worker = batch // NUM_WORKERS
    n_chunks = batches_per_worker // BATCH_CHUNK
    mesh = plsc.VectorSubcoreMesh(core_axis_name="c", subcore_axis_name="s")

    @functools.partial(
        pl.kernel,
        mesh=mesh,
        out_type=jax.ShapeDtypeStruct((batch, hist, D_MODEL), jnp.float32),
        scratch_types=[
            pltpu.VMEM((rows,), jnp.int32),
            pltpu.VMEM((rows, D_MODEL), jnp.float32),
            pltpu.SemaphoreType.DMA,
        ],
        compiler_params=pltpu.CompilerParams(use_tc_tiling_on_sc=False),
    )
    def gather_kernel(idx_hbm, table_hbm, out_hbm, idx_v, rows_v, sem):
        wid = lax.axis_index("s") * 2 + lax.axis_index("c")
        base_b = wid * batches_per_worker

        def body(i, carry):
            b = base_b + i * BATCH_CHUNK
            pltpu.sync_copy(idx_hbm.at[pl.ds(b * hist, rows)], idx_v)
            pltpu.async_copy(table_hbm.at[idx_v], rows_v, sem).wait()
            for j in range(BATCH_CHUNK):
                pltpu.sync_copy(
                    rows_v.at[pl.ds(j * hist, hist)], out_hbm.at[b + j]
                )
            return carry

        lax.fori_loop(0, n_chunks, body, 0)

    return gather_kernel


def kernel(x, table):
    b, h = x.shape
    idx = x.reshape(-1).astype(jnp.int32)
    return _make_gather(b, h)(idx, table)
